# baseline (speedup 1.0000x reference)
import jax
import jax.numpy as jnp
from jax import lax
from jax.experimental import pallas as pl
from jax.experimental.pallas import tpu as pltpu

_HI = lax.Precision.HIGHEST
_LO = lax.Precision.DEFAULT


def _dot(a, b, precision=_HI):
    return jnp.dot(a, b, preferred_element_type=jnp.float32,
                   precision=precision)


def _dot_t(a, b):
    return lax.dot_general(a, b, (((1,), (1,)), ((), ())),
                           preferred_element_type=jnp.float32, precision=_HI)


def _layer_norm(y, g, b, eps=1e-5):
    mean = jnp.mean(y, axis=-1, keepdims=True)
    var = jnp.mean((y - mean) ** 2, axis=-1, keepdims=True)
    return (y - mean) * lax.rsqrt(var + eps) * g + b


def _decoder_kernel(ids_sref, dpadp_ref, epadp_ref, emb_ref, pe_ref, enc_ref,
                    sqkvw_r, sqkvb_r, swo_r, sbo_r,
                    slg_r, slb_r,
                    cwq_r, cbq_r, ckvw_r, ckvb_r, cwo_r, cbo_r,
                    clg_r, clb_r,
                    fw1_r, fb1_r, fw2_r, fb2_r, flg_r, flb_r,
                    out_ref, sa0_r, ca0_r, sa1_r, ca1_r,
                    q_s, k_s, v_s, ctx_s, kbd_s, vbd_s):
    L, D = pe_ref.shape
    V = emb_ref.shape[0]
    H = 8
    hd = D // H
    rph = L // H
    nL = swo_r.shape[0]
    scale = float(hd) ** 0.5

    b = pl.program_id(0)

    def gather_body(i, c):
        tok = ids_sref[b * L + i]
        q_s[pl.ds(i, 1), :] = emb_ref[pl.ds(tok, 1), :]
        return c

    lax.fori_loop(0, L, gather_body, 0, unroll=8)
    x = q_s[...] + pe_ref[...]

    rowp = lax.broadcasted_iota(jnp.int32, (L, L), 0)
    colp = lax.broadcasted_iota(jnp.int32, (L, L), 1)
    ltrue_row = 8 * (rowp % rph) + rowp // rph
    ltrue_col = 8 * (colp % rph) + colp // rph
    causal_p = ltrue_col > ltrue_row
    gperm = (colp == rph * (rowp % H) + rowp // H).astype(jnp.float32)
    neg_inf = jnp.float32(-jnp.inf)

    enc = enc_ref[0]

    kbd_s[...] = jnp.zeros_like(kbd_s)
    vbd_s[...] = jnp.zeros_like(vbd_s)

    def head_perm(src_ref, h):
        blk = src_ref[pl.ds(rph * h, rph), :]
        return jnp.concatenate(
            [blk[:, hd * c:hd * (c + 1)] for c in range(H)], axis=0)

    def attention(xq, wo, bo, g, beta,
                  pad_ref, use_causal, attn_ref, post_prec):

        ps = []
        for grp in range(2):
            q4 = jnp.concatenate(
                [head_perm(q_s, 4 * grp + a) for a in range(4)], axis=1)
            for a in range(4):
                kbd_s[pl.ds(L * a, L), hd * a:hd * (a + 1)] = \
                    head_perm(k_s, 4 * grp + a)
            s4 = lax.dot_general(q4, kbd_s[...], (((1,), (1,)), ((), ())),
                                 preferred_element_type=jnp.float32,
                                 precision=_HI)
            s4 = jnp.floor(s4 / scale)
            for a in range(4):
                h = 4 * grp + a
                s = s4[:, L * a:L * (a + 1)]
                pad = pad_ref[h:h + 1, :]
                mask = jnp.broadcast_to(pad != 0, (L, L))
                if use_causal:
                    mask = mask | causal_p
                s = jnp.where(mask, neg_inf, s)
                m = jnp.max(s, axis=-1, keepdims=True)
                e = jnp.exp(s - m)
                p = e / jnp.sum(e, axis=-1, keepdims=True)
                pt = lax.dot_general(_dot(gperm, p, precision=_LO), gperm,
                                     (((1,), (1,)), ((), ())),
                                     preferred_element_type=jnp.float32,
                                     precision=_LO)
                attn_ref[h, :, :] = pt
                ps.append(p)

        for pair in range(4):
            h0 = 2 * pair
            p2 = jnp.concatenate([ps[h0], ps[h0 + 1]], axis=1)
            for a in range(2):
                vbd_s[pl.ds(L * a, L), hd * a:hd * (a + 1)] = \
                    head_perm(v_s, h0 + a)
            ctx2 = _dot(p2, vbd_s[...], precision=post_prec)
            for a in range(2):
                h = h0 + a
                ctx_h = ctx2[:, hd * a:hd * (a + 1)]
                ctx_s[pl.ds(rph * h, rph), :] = jnp.concatenate(
                    [ctx_h[rph * c:rph * (c + 1), :] for c in range(H)],
                    axis=1)

        o = _dot(ctx_s[...], wo, precision=post_prec) + bo
        return _layer_norm(xq + o, g, beta)

    sa_refs = (sa0_r, sa1_r)
    ca_refs = (ca0_r, ca1_r)
    for l in range(nL):
        last = l == nL - 1
        qkv = _dot(x, sqkvw_r[l]) + sqkvb_r[l:l + 1, :]
        q_s[...] = qkv[:, :D]
        k_s[...] = qkv[:, D:2 * D]
        v_s[...] = qkv[:, 2 * D:]
        x = attention(x, swo_r[l], sbo_r[l:l + 1, :],
                      slg_r[l:l + 1, :], slb_r[l:l + 1, :],
                      dpadp_ref, True, sa_refs[l], _HI)
        q_s[...] = _dot(x, cwq_r[l]) + cbq_r[l:l + 1, :]
        kv = _dot(enc, ckvw_r[l]) + ckvb_r[l:l + 1, :]
        k_s[...] = kv[:, :D]
        v_s[...] = kv[:, D:]
        x = attention(x, cwo_r[l], cbo_r[l:l + 1, :],
                      clg_r[l:l + 1, :], clb_r[l:l + 1, :],
                      epadp_ref, False, ca_refs[l], _LO if last else _HI)
        ffn_prec = _LO if last else _HI
        h1 = jnp.maximum(_dot(x, fw1_r[l], precision=ffn_prec)
                         + fb1_r[l:l + 1, :], 0.0)
        o = _dot(h1, fw2_r[l], precision=ffn_prec) + fb2_r[l:l + 1, :]
        x = _layer_norm(x + o, flg_r[l:l + 1, :], flb_r[l:l + 1, :])

    out_ref[0] = x


def kernel(word_emb, pos_emb, self_wq, self_bq, self_wk, self_bk, self_wv,
           self_bv, self_wo, self_bo, self_ln_g, self_ln_b,
           ctx_wq, ctx_bq, ctx_wk, ctx_bk, ctx_wv, ctx_bv, ctx_wo, ctx_bo,
           ctx_ln_g, ctx_ln_b,
           ffn_w1, ffn_b1, ffn_w2, ffn_b2, ffn_ln_g, ffn_ln_b,
           dec_ids, enc_ids, enc_out):
    B, L = dec_ids.shape
    V, D = word_emb.shape
    nL = self_wq.shape[0]
    H = 8
    rph = L // H

    ids_flat = dec_ids.reshape(B * L).astype(jnp.int32)
    pe_rows = pos_emb[:L]
    dpad = (dec_ids == 0).astype(jnp.int32)
    epad = (enc_ids == 0).astype(jnp.int32)
    dpad_p = dpad.reshape(B, rph, H).transpose(0, 2, 1).reshape(B, L)
    epad_p = epad.reshape(B, rph, H).transpose(0, 2, 1).reshape(B, L)

    def _const2(a):
        return pl.BlockSpec(a.shape, lambda b, s: (0,) * a.ndim)

    sqkv_w = jnp.concatenate([self_wq, self_wk, self_wv], axis=2)
    sqkv_b = jnp.concatenate([self_bq, self_bk, self_bv], axis=1)
    ckv_w = jnp.concatenate([ctx_wk, ctx_wv], axis=2)
    ckv_b = jnp.concatenate([ctx_bk, ctx_bv], axis=1)
    weights = [sqkv_w, sqkv_b, self_wo, self_bo, self_ln_g, self_ln_b,
               ctx_wq, ctx_bq, ckv_w, ckv_b,
               ctx_wo, ctx_bo, ctx_ln_g, ctx_ln_b,
               ffn_w1, ffn_b1, ffn_w2, ffn_b2, ffn_ln_g, ffn_ln_b]

    in_specs = [
        _const2(dpad_p),
        _const2(epad_p),
        _const2(word_emb),
        _const2(pe_rows),
        pl.BlockSpec((1, L, D), lambda b, s: (b, 0, 0)),
    ] + [_const2(w) for w in weights]

    out_shape = (
        jax.ShapeDtypeStruct((B, L, D), jnp.float32),
        jax.ShapeDtypeStruct((B * H, L, L), jnp.float32),
        jax.ShapeDtypeStruct((B * H, L, L), jnp.float32),
        jax.ShapeDtypeStruct((B * H, L, L), jnp.float32),
        jax.ShapeDtypeStruct((B * H, L, L), jnp.float32),
    )
    out_specs = [
        pl.BlockSpec((1, L, D), lambda b, s: (b, 0, 0)),
        pl.BlockSpec((H, L, L), lambda b, s: (b, 0, 0)),
        pl.BlockSpec((H, L, L), lambda b, s: (b, 0, 0)),
        pl.BlockSpec((H, L, L), lambda b, s: (b, 0, 0)),
        pl.BlockSpec((H, L, L), lambda b, s: (b, 0, 0)),
    ]

    out, sa0, ca0, sa1, ca1 = pl.pallas_call(
        _decoder_kernel,
        out_shape=out_shape,
        grid_spec=pltpu.PrefetchScalarGridSpec(
            num_scalar_prefetch=1,
            grid=(B,),
            in_specs=in_specs,
            out_specs=out_specs,
            scratch_shapes=[pltpu.VMEM((L, D), jnp.float32)] * 4
            + [pltpu.VMEM((4 * L, 4 * (D // H)), jnp.float32),
               pltpu.VMEM((2 * L, 2 * (D // H)), jnp.float32)],
        ),
        compiler_params=pltpu.CompilerParams(
            dimension_semantics=("arbitrary",),
            vmem_limit_bytes=64 * 1024 * 1024,
        ),
    )(ids_flat, dpad_p, epad_p, word_emb, pe_rows, enc_out, *weights)

    return out, [sa0, sa1], [ca0, ca1]


# speedup vs baseline: 1.0627x; 1.0627x over previous
import jax
import jax.numpy as jnp
from jax import lax
from jax.experimental import pallas as pl
from jax.experimental.pallas import tpu as pltpu

_HI = lax.Precision.HIGHEST
_LO = lax.Precision.DEFAULT


def _dot(a, b, precision=_HI):
    return jnp.dot(a, b, preferred_element_type=jnp.float32,
                   precision=precision)


def _dot_t(a, b):
    return lax.dot_general(a, b, (((1,), (1,)), ((), ())),
                           preferred_element_type=jnp.float32, precision=_HI)


def _layer_norm(y, g, b, eps=1e-5):
    mean = jnp.mean(y, axis=-1, keepdims=True)
    var = jnp.mean((y - mean) ** 2, axis=-1, keepdims=True)
    return (y - mean) * lax.rsqrt(var + eps) * g + b


def _decoder_kernel(ids_sref, dpadp_ref, epadp_ref, emb_ref, pe_ref, enc_ref,
                    swq_r, sbq_r, swk_r, sbk_r, swv_r, sbv_r, swo_r, sbo_r,
                    slg_r, slb_r,
                    cwq_r, cbq_r, cwk_r, cbk_r, cwv_r, cbv_r, cwo_r, cbo_r,
                    clg_r, clb_r,
                    fw1_r, fb1_r, fw2_r, fb2_r, flg_r, flb_r,
                    out_ref, sa0_r, ca0_r, sa1_r, ca1_r,
                    q_s, k_s, v_s, ctx_s, kbd_s, vbd_s):
    L, D = pe_ref.shape
    V = emb_ref.shape[0]
    H = 8
    hd = D // H
    rph = L // H
    nL = swq_r.shape[0]
    scale = float(hd) ** 0.5

    b = pl.program_id(0)

    def gather_body(i, c):
        tok = ids_sref[b * L + i]
        q_s[pl.ds(i, 1), :] = emb_ref[pl.ds(tok, 1), :]
        return c

    lax.fori_loop(0, L, gather_body, 0, unroll=8)
    x = q_s[...] + pe_ref[...]

    rowp = lax.broadcasted_iota(jnp.int32, (L, L), 0)
    colp = lax.broadcasted_iota(jnp.int32, (L, L), 1)
    ltrue_row = 8 * (rowp % rph) + rowp // rph
    ltrue_col = 8 * (colp % rph) + colp // rph
    causal_p = ltrue_col > ltrue_row
    gperm = (colp == rph * (rowp % H) + rowp // H).astype(jnp.float32)
    neg_inf = jnp.float32(-jnp.inf)

    enc = enc_ref[0]

    kbd_s[...] = jnp.zeros_like(kbd_s)
    vbd_s[...] = jnp.zeros_like(vbd_s)

    def head_perm(src_ref, h):
        blk = src_ref[pl.ds(rph * h, rph), :]
        return jnp.concatenate(
            [blk[:, hd * c:hd * (c + 1)] for c in range(H)], axis=0)

    def attention(xq, key_val, wq, bq, wk, bk, wv, bv, wo, bo, g, beta,
                  pad_ref, use_causal, attn_ref, post_prec):
        q_s[...] = _dot(xq, wq) + bq
        k_s[...] = _dot(key_val, wk) + bk
        v_s[...] = _dot(key_val, wv, precision=post_prec) + bv

        ps = []
        for grp in range(2):
            q4 = jnp.concatenate(
                [head_perm(q_s, 4 * grp + a) for a in range(4)], axis=1)
            for a in range(4):
                kbd_s[pl.ds(L * a, L), hd * a:hd * (a + 1)] = \
                    head_perm(k_s, 4 * grp + a)
            s4 = lax.dot_general(q4, kbd_s[...], (((1,), (1,)), ((), ())),
                                 preferred_element_type=jnp.float32,
                                 precision=_HI)
            s4 = jnp.floor(s4 / scale)
            for a in range(4):
                h = 4 * grp + a
                s = s4[:, L * a:L * (a + 1)]
                pad = pad_ref[h:h + 1, :]
                mask = jnp.broadcast_to(pad != 0, (L, L))
                if use_causal:
                    mask = mask | causal_p
                s = jnp.where(mask, neg_inf, s)
                m = jnp.max(s, axis=-1, keepdims=True)
                e = jnp.exp(s - m)
                p = e / jnp.sum(e, axis=-1, keepdims=True)
                pt = lax.dot_general(_dot(gperm, p, precision=_LO), gperm,
                                     (((1,), (1,)), ((), ())),
                                     preferred_element_type=jnp.float32,
                                     precision=_LO)
                attn_ref[h, :, :] = pt
                ps.append(p)

        for pair in range(4):
            h0 = 2 * pair
            p2 = jnp.concatenate([ps[h0], ps[h0 + 1]], axis=1)
            for a in range(2):
                vbd_s[pl.ds(L * a, L), hd * a:hd * (a + 1)] = \
                    head_perm(v_s, h0 + a)
            ctx2 = _dot(p2, vbd_s[...], precision=post_prec)
            for a in range(2):
                h = h0 + a
                ctx_h = ctx2[:, hd * a:hd * (a + 1)]
                ctx_s[pl.ds(rph * h, rph), :] = jnp.concatenate(
                    [ctx_h[rph * c:rph * (c + 1), :] for c in range(H)],
                    axis=1)

        o = _dot(ctx_s[...], wo, precision=post_prec) + bo
        return _layer_norm(xq + o, g, beta)

    sa_refs = (sa0_r, sa1_r)
    ca_refs = (ca0_r, ca1_r)
    for l in range(nL):
        last = l == nL - 1
        x = attention(x, x,
                      swq_r[l], sbq_r[l:l + 1, :], swk_r[l], sbk_r[l:l + 1, :],
                      swv_r[l], sbv_r[l:l + 1, :], swo_r[l], sbo_r[l:l + 1, :],
                      slg_r[l:l + 1, :], slb_r[l:l + 1, :],
                      dpadp_ref, True, sa_refs[l], _HI)
        x = attention(x, enc,
                      cwq_r[l], cbq_r[l:l + 1, :], cwk_r[l], cbk_r[l:l + 1, :],
                      cwv_r[l], cbv_r[l:l + 1, :], cwo_r[l], cbo_r[l:l + 1, :],
                      clg_r[l:l + 1, :], clb_r[l:l + 1, :],
                      epadp_ref, False, ca_refs[l], _LO if last else _HI)
        ffn_prec = _LO if last else _HI
        h1 = jnp.maximum(_dot(x, fw1_r[l], precision=ffn_prec)
                         + fb1_r[l:l + 1, :], 0.0)
        o = _dot(h1, fw2_r[l], precision=ffn_prec) + fb2_r[l:l + 1, :]
        x = _layer_norm(x + o, flg_r[l:l + 1, :], flb_r[l:l + 1, :])

    out_ref[0] = x


def kernel(word_emb, pos_emb, self_wq, self_bq, self_wk, self_bk, self_wv,
           self_bv, self_wo, self_bo, self_ln_g, self_ln_b,
           ctx_wq, ctx_bq, ctx_wk, ctx_bk, ctx_wv, ctx_bv, ctx_wo, ctx_bo,
           ctx_ln_g, ctx_ln_b,
           ffn_w1, ffn_b1, ffn_w2, ffn_b2, ffn_ln_g, ffn_ln_b,
           dec_ids, enc_ids, enc_out):
    B, L = dec_ids.shape
    V, D = word_emb.shape
    nL = self_wq.shape[0]
    H = 8
    rph = L // H

    ids_flat = dec_ids.reshape(B * L).astype(jnp.int32)
    pe_rows = pos_emb[:L]
    dpad = (dec_ids == 0).astype(jnp.int32)
    epad = (enc_ids == 0).astype(jnp.int32)
    dpad_p = dpad.reshape(B, rph, H).transpose(0, 2, 1).reshape(B, L)
    epad_p = epad.reshape(B, rph, H).transpose(0, 2, 1).reshape(B, L)

    def _const2(a):
        return pl.BlockSpec(a.shape, lambda b, s: (0,) * a.ndim)

    weights = [self_wq, self_bq, self_wk, self_bk, self_wv, self_bv,
               self_wo, self_bo, self_ln_g, self_ln_b,
               ctx_wq, ctx_bq, ctx_wk, ctx_bk, ctx_wv, ctx_bv,
               ctx_wo, ctx_bo, ctx_ln_g, ctx_ln_b,
               ffn_w1, ffn_b1, ffn_w2, ffn_b2, ffn_ln_g, ffn_ln_b]

    in_specs = [
        _const2(dpad_p),
        _const2(epad_p),
        _const2(word_emb),
        _const2(pe_rows),
        pl.BlockSpec((1, L, D), lambda b, s: (b, 0, 0)),
    ] + [_const2(w) for w in weights]

    out_shape = (
        jax.ShapeDtypeStruct((B, L, D), jnp.float32),
        jax.ShapeDtypeStruct((B * H, L, L), jnp.float32),
        jax.ShapeDtypeStruct((B * H, L, L), jnp.float32),
        jax.ShapeDtypeStruct((B * H, L, L), jnp.float32),
        jax.ShapeDtypeStruct((B * H, L, L), jnp.float32),
    )
    out_specs = [
        pl.BlockSpec((1, L, D), lambda b, s: (b, 0, 0)),
        pl.BlockSpec((H, L, L), lambda b, s: (b, 0, 0)),
        pl.BlockSpec((H, L, L), lambda b, s: (b, 0, 0)),
        pl.BlockSpec((H, L, L), lambda b, s: (b, 0, 0)),
        pl.BlockSpec((H, L, L), lambda b, s: (b, 0, 0)),
    ]

    out, sa0, ca0, sa1, ca1 = pl.pallas_call(
        _decoder_kernel,
        out_shape=out_shape,
        grid_spec=pltpu.PrefetchScalarGridSpec(
            num_scalar_prefetch=1,
            grid=(B,),
            in_specs=in_specs,
            out_specs=out_specs,
            scratch_shapes=[pltpu.VMEM((L, D), jnp.float32)] * 4
            + [pltpu.VMEM((4 * L, 4 * (D // H)), jnp.float32),
               pltpu.VMEM((2 * L, 2 * (D // H)), jnp.float32)],
        ),
        compiler_params=pltpu.CompilerParams(
            dimension_semantics=("arbitrary",),
            vmem_limit_bytes=64 * 1024 * 1024,
        ),
    )(ids_flat, dpad_p, epad_p, word_emb, pe_rows, enc_out, *weights)

    return out, [sa0, sa1], [ca0, ca1]


# speedup vs baseline: 1.0663x; 1.0034x over previous
import jax
import jax.numpy as jnp
from jax import lax
from jax.experimental import pallas as pl
from jax.experimental.pallas import tpu as pltpu

_HI = lax.Precision.HIGHEST
_LO = lax.Precision.DEFAULT


def _dot(a, b, precision=_HI):
    return jnp.dot(a, b, preferred_element_type=jnp.float32,
                   precision=precision)


def _dot_t(a, b):
    return lax.dot_general(a, b, (((1,), (1,)), ((), ())),
                           preferred_element_type=jnp.float32, precision=_HI)


def _layer_norm(y, g, b, eps=1e-5):
    mean = jnp.mean(y, axis=-1, keepdims=True)
    var = jnp.mean((y - mean) ** 2, axis=-1, keepdims=True)
    return (y - mean) * lax.rsqrt(var + eps) * g + b


def _decoder_kernel(ids_sref, dpadp_ref, epadp_ref, emb_ref, pe_ref, enc_ref,
                    swq_r, sbq_r, swk_r, sbk_r, swv_r, sbv_r, swo_r, sbo_r,
                    slg_r, slb_r,
                    cwq_r, cbq_r, cwk_r, cbk_r, cwv_r, cbv_r, cwo_r, cbo_r,
                    clg_r, clb_r,
                    fw1_r, fb1_r, fw2_r, fb2_r, flg_r, flb_r,
                    out_ref, sa0_r, ca0_r, sa1_r, ca1_r,
                    q_s, k_s, v_s, ctx_s, kbd_s, vbd_s):
    L, D = pe_ref.shape
    V = emb_ref.shape[0]
    H = 8
    hd = D // H
    rph = L // H
    nL = swq_r.shape[0]
    scale = float(hd) ** 0.5

    b = pl.program_id(0)

    def gather_body(i, c):
        tok = ids_sref[b * L + i]
        q_s[pl.ds(i, 1), :] = emb_ref[pl.ds(tok, 1), :]
        return c

    lax.fori_loop(0, L, gather_body, 0, unroll=16)
    x = q_s[...] + pe_ref[...]

    rowp = lax.broadcasted_iota(jnp.int32, (L, L), 0)
    colp = lax.broadcasted_iota(jnp.int32, (L, L), 1)
    ltrue_row = 8 * (rowp % rph) + rowp // rph
    ltrue_col = 8 * (colp % rph) + colp // rph
    causal_p = ltrue_col > ltrue_row
    gperm = (colp == rph * (rowp % H) + rowp // H).astype(jnp.float32)
    neg_inf = jnp.float32(-jnp.inf)

    enc = enc_ref[0]

    kbd_s[...] = jnp.zeros_like(kbd_s)
    vbd_s[...] = jnp.zeros_like(vbd_s)

    def head_perm(src_ref, h):
        blk = src_ref[pl.ds(rph * h, rph), :]
        return jnp.concatenate(
            [blk[:, hd * c:hd * (c + 1)] for c in range(H)], axis=0)

    def attention(xq, key_val, wq, bq, wk, bk, wv, bv, wo, bo, g, beta,
                  pad_ref, use_causal, attn_ref, post_prec):
        q_s[...] = _dot(xq, wq) + bq
        k_s[...] = _dot(key_val, wk) + bk
        v_s[...] = _dot(key_val, wv, precision=post_prec) + bv

        ps = []
        for grp in range(2):
            q4 = jnp.concatenate(
                [head_perm(q_s, 4 * grp + a) for a in range(4)], axis=1)
            for a in range(4):
                kbd_s[pl.ds(L * a, L), hd * a:hd * (a + 1)] = \
                    head_perm(k_s, 4 * grp + a)
            s4 = lax.dot_general(q4, kbd_s[...], (((1,), (1,)), ((), ())),
                                 preferred_element_type=jnp.float32,
                                 precision=_HI)
            s4 = jnp.floor(s4 / scale)
            for a in range(4):
                h = 4 * grp + a
                s = s4[:, L * a:L * (a + 1)]
                pad = pad_ref[h:h + 1, :]
                mask = jnp.broadcast_to(pad != 0, (L, L))
                if use_causal:
                    mask = mask | causal_p
                s = jnp.where(mask, neg_inf, s)
                m = jnp.max(s, axis=-1, keepdims=True)
                e = jnp.exp(s - m)
                p = e / jnp.sum(e, axis=-1, keepdims=True)
                pt = lax.dot_general(_dot(gperm, p, precision=_LO), gperm,
                                     (((1,), (1,)), ((), ())),
                                     preferred_element_type=jnp.float32,
                                     precision=_LO)
                attn_ref[h, :, :] = pt
                ps.append(p)

        for pair in range(4):
            h0 = 2 * pair
            p2 = jnp.concatenate([ps[h0], ps[h0 + 1]], axis=1)
            for a in range(2):
                vbd_s[pl.ds(L * a, L), hd * a:hd * (a + 1)] = \
                    head_perm(v_s, h0 + a)
            ctx2 = _dot(p2, vbd_s[...], precision=post_prec)
            for a in range(2):
                h = h0 + a
                ctx_h = ctx2[:, hd * a:hd * (a + 1)]
                ctx_s[pl.ds(rph * h, rph), :] = jnp.concatenate(
                    [ctx_h[rph * c:rph * (c + 1), :] for c in range(H)],
                    axis=1)

        o = _dot(ctx_s[...], wo, precision=post_prec) + bo
        return _layer_norm(xq + o, g, beta)

    sa_refs = (sa0_r, sa1_r)
    ca_refs = (ca0_r, ca1_r)
    for l in range(nL):
        last = l == nL - 1
        x = attention(x, x,
                      swq_r[l], sbq_r[l:l + 1, :], swk_r[l], sbk_r[l:l + 1, :],
                      swv_r[l], sbv_r[l:l + 1, :], swo_r[l], sbo_r[l:l + 1, :],
                      slg_r[l:l + 1, :], slb_r[l:l + 1, :],
                      dpadp_ref, True, sa_refs[l], _HI)
        x = attention(x, enc,
                      cwq_r[l], cbq_r[l:l + 1, :], cwk_r[l], cbk_r[l:l + 1, :],
                      cwv_r[l], cbv_r[l:l + 1, :], cwo_r[l], cbo_r[l:l + 1, :],
                      clg_r[l:l + 1, :], clb_r[l:l + 1, :],
                      epadp_ref, False, ca_refs[l], _LO if last else _HI)
        ffn_prec = _LO if last else _HI
        h1 = jnp.maximum(_dot(x, fw1_r[l], precision=ffn_prec)
                         + fb1_r[l:l + 1, :], 0.0)
        o = _dot(h1, fw2_r[l], precision=ffn_prec) + fb2_r[l:l + 1, :]
        x = _layer_norm(x + o, flg_r[l:l + 1, :], flb_r[l:l + 1, :])

    out_ref[0] = x


def kernel(word_emb, pos_emb, self_wq, self_bq, self_wk, self_bk, self_wv,
           self_bv, self_wo, self_bo, self_ln_g, self_ln_b,
           ctx_wq, ctx_bq, ctx_wk, ctx_bk, ctx_wv, ctx_bv, ctx_wo, ctx_bo,
           ctx_ln_g, ctx_ln_b,
           ffn_w1, ffn_b1, ffn_w2, ffn_b2, ffn_ln_g, ffn_ln_b,
           dec_ids, enc_ids, enc_out):
    B, L = dec_ids.shape
    V, D = word_emb.shape
    nL = self_wq.shape[0]
    H = 8
    rph = L // H

    ids_flat = dec_ids.reshape(B * L).astype(jnp.int32)
    pe_rows = pos_emb[:L]
    dpad = (dec_ids == 0).astype(jnp.int32)
    epad = (enc_ids == 0).astype(jnp.int32)
    dpad_p = dpad.reshape(B, rph, H).transpose(0, 2, 1).reshape(B, L)
    epad_p = epad.reshape(B, rph, H).transpose(0, 2, 1).reshape(B, L)

    def _const2(a):
        return pl.BlockSpec(a.shape, lambda b, s: (0,) * a.ndim)

    weights = [self_wq, self_bq, self_wk, self_bk, self_wv, self_bv,
               self_wo, self_bo, self_ln_g, self_ln_b,
               ctx_wq, ctx_bq, ctx_wk, ctx_bk, ctx_wv, ctx_bv,
               ctx_wo, ctx_bo, ctx_ln_g, ctx_ln_b,
               ffn_w1, ffn_b1, ffn_w2, ffn_b2, ffn_ln_g, ffn_ln_b]

    in_specs = [
        _const2(dpad_p),
        _const2(epad_p),
        _const2(word_emb),
        _const2(pe_rows),
        pl.BlockSpec((1, L, D), lambda b, s: (b, 0, 0)),
    ] + [_const2(w) for w in weights]

    out_shape = (
        jax.ShapeDtypeStruct((B, L, D), jnp.float32),
        jax.ShapeDtypeStruct((B * H, L, L), jnp.float32),
        jax.ShapeDtypeStruct((B * H, L, L), jnp.float32),
        jax.ShapeDtypeStruct((B * H, L, L), jnp.float32),
        jax.ShapeDtypeStruct((B * H, L, L), jnp.float32),
    )
    out_specs = [
        pl.BlockSpec((1, L, D), lambda b, s: (b, 0, 0)),
        pl.BlockSpec((H, L, L), lambda b, s: (b, 0, 0)),
        pl.BlockSpec((H, L, L), lambda b, s: (b, 0, 0)),
        pl.BlockSpec((H, L, L), lambda b, s: (b, 0, 0)),
        pl.BlockSpec((H, L, L), lambda b, s: (b, 0, 0)),
    ]

    out, sa0, ca0, sa1, ca1 = pl.pallas_call(
        _decoder_kernel,
        out_shape=out_shape,
        grid_spec=pltpu.PrefetchScalarGridSpec(
            num_scalar_prefetch=1,
            grid=(B,),
            in_specs=in_specs,
            out_specs=out_specs,
            scratch_shapes=[pltpu.VMEM((L, D), jnp.float32)] * 4
            + [pltpu.VMEM((4 * L, 4 * (D // H)), jnp.float32),
               pltpu.VMEM((2 * L, 2 * (D // H)), jnp.float32)],
        ),
        compiler_params=pltpu.CompilerParams(
            dimension_semantics=("arbitrary",),
            vmem_limit_bytes=64 * 1024 * 1024,
        ),
    )(ids_flat, dpad_p, epad_p, word_emb, pe_rows, enc_out, *weights)

    return out, [sa0, sa1], [ca0, ca1]
